# baseline (speedup 1.0000x reference)
import functools

import jax
import jax.numpy as jnp
from jax import lax
from jax.experimental import pallas as pl
from jax.experimental.pallas import tpu as pltpu
from jax.experimental.pallas import tpu_sc as plsc

N = 10000
NP = 10240
E = 320000
D = 128
G = 80
NC = 2
NS = 16
NW = NC * NS
CPW = E // (G * NW)
RPS = NP // NS

BM = 1000


def _mesh():
    return plsc.VectorSubcoreMesh(core_axis_name="c", subcore_axis_name="s")


def _fill_rows(ref, val):
    v = jnp.full((16,), val, jnp.float32)
    rows = ref.shape[0]

    def body(i, c):
        for j in range(D // 16):
            ref[i, pl.ds(j * 16, 16)] = v
        return c

    lax.fori_loop(0, rows, body, 0)


def _zero_acc_async(zbuf, acc, sid, sem):
    for k2 in range(RPS // G):
        pltpu.async_copy(zbuf, acc.at[pl.ds(sid * RPS + k2 * G, G)], sem)


def _zero_acc_wait(zbuf, acc, sid, sem):
    for k2 in range(RPS // G):
        pltpu.make_async_copy(
            zbuf, acc.at[pl.ds(sid * RPS + k2 * G, G)], sem).wait()


def _sc_hist(col2d):
    @functools.partial(
        pl.kernel,
        out_type=jax.ShapeDtypeStruct((NC, NP, D), jnp.float32),
        mesh=_mesh(),
        scratch_types=[
            pltpu.VMEM((CPW, G), jnp.int32),
            pltpu.VMEM((G, D), jnp.float32),
            pltpu.VMEM((G, D), jnp.float32),
            pltpu.VMEM_SHARED((NP, D), jnp.float32),
            pltpu.SemaphoreType.DMA,
            pltpu.SemaphoreType.DMA,
        ],
    )
    def k(col_hbm, out_hbm, cidx_v, ones_v, zbuf, acc, semi, semz):
        cid = lax.axis_index("c")
        sid = lax.axis_index("s")
        wid = sid * NC + cid
        pltpu.async_copy(col_hbm.at[wid], cidx_v, semi)
        _fill_rows(zbuf, 0.0)
        _zero_acc_async(zbuf, acc, sid, semz)
        _fill_rows(ones_v, 1.0)
        pltpu.make_async_copy(col_hbm.at[wid], cidx_v, semi).wait()
        _zero_acc_wait(zbuf, acc, sid, semz)
        plsc.subcore_barrier()

        def body(g, c):
            pltpu.sync_copy(ones_v, acc.at[cidx_v.at[g]], add=True)
            return c

        lax.fori_loop(0, CPW, body, 0)
        plsc.subcore_barrier()
        pltpu.sync_copy(acc.at[pl.ds(sid * RPS, RPS)],
                        out_hbm.at[cid, pl.ds(sid * RPS, RPS)])

    return k(col2d)


def _sc_prop(row1d, col2d, y):
    @functools.partial(
        pl.kernel,
        out_type=jax.ShapeDtypeStruct((NC, NP, D), jnp.float32),
        mesh=_mesh(),
        scratch_types=[
            pltpu.VMEM((CPW * G,), jnp.int32),
            pltpu.VMEM((CPW, G), jnp.int32),
            pltpu.VMEM((G, D), jnp.float32),
            pltpu.VMEM((G, D), jnp.float32),
            pltpu.VMEM_SHARED((NP, D), jnp.float32),
            pltpu.SemaphoreType.DMA,
            pltpu.SemaphoreType.DMA,
            pltpu.SemaphoreType.DMA,
        ],
    )
    def k(row_hbm, col_hbm, y_hbm, out_hbm,
          ridx_v, cidx_v, buf0, buf1, acc, sem0, sem1, semz):
        cid = lax.axis_index("c")
        sid = lax.axis_index("s")
        wid = sid * NC + cid
        pltpu.async_copy(row_hbm.at[pl.ds(wid * (CPW * G), CPW * G)],
                         ridx_v, sem0)
        pltpu.async_copy(col_hbm.at[wid], cidx_v, sem1)
        _fill_rows(buf0, 0.0)
        _zero_acc_async(buf0, acc, sid, semz)
        pltpu.make_async_copy(row_hbm.at[pl.ds(wid * (CPW * G), CPW * G)],
                              ridx_v, sem0).wait()
        pltpu.make_async_copy(col_hbm.at[wid], cidx_v, sem1).wait()
        _zero_acc_wait(buf0, acc, sid, semz)
        plsc.subcore_barrier()

        H = G // 2

        def gfire(g, buf, sem):
            pltpu.async_copy(y_hbm.at[ridx_v.at[pl.ds(g * G, H)]],
                             buf.at[pl.ds(0, H)], sem)
            pltpu.async_copy(y_hbm.at[ridx_v.at[pl.ds(g * G + H, H)]],
                             buf.at[pl.ds(H, H)], sem)

        def gwait(g, buf, sem):
            pltpu.make_async_copy(y_hbm.at[ridx_v.at[pl.ds(g * G, H)]],
                                  buf.at[pl.ds(0, H)], sem).wait()
            pltpu.make_async_copy(y_hbm.at[ridx_v.at[pl.ds(g * G + H, H)]],
                                  buf.at[pl.ds(H, H)], sem).wait()

        gfire(0, buf0, sem0)

        def body(gg, c):
            g = 2 * gg
            gwait(g, buf0, sem0)
            gfire(g + 1, buf1, sem1)
            pltpu.sync_copy(buf0, acc.at[cidx_v.at[g]], add=True)
            gwait(g + 1, buf1, sem1)
            gfire(g + 2, buf0, sem0)
            pltpu.sync_copy(buf1, acc.at[cidx_v.at[g + 1]], add=True)
            return c

        lax.fori_loop(0, CPW // 2, body, 0)
        gwait(CPW - 1, buf0, sem0)
        pltpu.sync_copy(buf0, acc.at[cidx_v.at[CPW - 1]], add=True)
        plsc.subcore_barrier()
        pltpu.sync_copy(acc.at[pl.ds(sid * RPS, RPS)],
                        out_hbm.at[cid, pl.ds(sid * RPS, RPS)])

    return k(row1d, col2d, y)


_ROW = pl.BlockSpec((BM, D), lambda i: (i, 0))
_PARTA = pl.BlockSpec((1, BM, D), lambda i: (0, i, 0))
_PARTB = pl.BlockSpec((1, BM, D), lambda i: (1, i, 0))
_DEG = pl.BlockSpec((BM, 1), lambda i: (i, 0))
_WFULL = pl.BlockSpec((D, D), lambda i: (0, 0))
_BIAS = pl.BlockSpec((1, D), lambda i: (0, 0))
_FLAG = pl.BlockSpec((1, 1), lambda i: (0, 0), memory_space=pltpu.SMEM)


def _deg_of(d_ref):
    return d_ref[...]


def _linear_l2(m, w, b, f):
    t = lax.dot_general(m, w, (((1,), (1,)), ((), ())),
                        preferred_element_type=jnp.float32) + b
    nrm = jnp.sqrt(jnp.sum(t * t, axis=1, keepdims=True))
    tn = t / jnp.maximum(nrm, 1e-12)
    return jnp.where(f == 1, tn, t)


def _dense1(h, deg):
    def body(h_ref, d_ref, o_ref):
        o_ref[...] = h_ref[...] * lax.rsqrt(_deg_of(d_ref))

    return pl.pallas_call(
        body,
        grid=(N // BM,),
        in_specs=[_ROW, _DEG],
        out_specs=_ROW,
        out_shape=jax.ShapeDtypeStruct((N, D), jnp.float32),
    )(h, deg)


def _dense2(p1, h, deg):
    def body(pa_ref, pb_ref, h_ref, d_ref, xo_ref, yo_ref):
        d = _deg_of(d_ref)
        s = lax.rsqrt(d)
        x = s * (pa_ref[0] + pb_ref[0]) + h_ref[...] / d
        xo_ref[...] = x
        yo_ref[...] = s * x

    return pl.pallas_call(
        body,
        grid=(N // BM,),
        in_specs=[_PARTA, _PARTB, _ROW, _DEG],
        out_specs=[_ROW, _ROW],
        out_shape=[jax.ShapeDtypeStruct((N, D), jnp.float32),
                   jax.ShapeDtypeStruct((N, D), jnp.float32)],
    )(p1, p1, h, deg)


def _final01(h, x1, W0, b0, W1, b1, flag):
    def body(h_ref, x1_ref, w0_ref, b0_ref, w1_ref, b1_ref, f_ref, o_ref):
        f = f_ref[0, 0]
        o_ref[:, 0:D] = _linear_l2(h_ref[...], w0_ref[...], b0_ref[...], f)
        o_ref[:, D:2 * D] = _linear_l2(x1_ref[...], w1_ref[...],
                                       b1_ref[...], f)

    return pl.pallas_call(
        body,
        grid=(N // BM,),
        in_specs=[_ROW, _ROW, _WFULL, _BIAS, _WFULL, _BIAS, _FLAG],
        out_specs=pl.BlockSpec((BM, 2 * D), lambda i: (i, 0)),
        out_shape=jax.ShapeDtypeStruct((N, 2 * D), jnp.float32),
    )(h, x1, W0, b0, W1, b1, flag)


def _final2(p2, x1, deg, W2, b2, flag):
    def body(pa_ref, pb_ref, x1_ref, d_ref,
             w2_ref, b2_ref, f_ref, o_ref):
        d = _deg_of(d_ref)
        s = lax.rsqrt(d)
        x2 = s * (pa_ref[0] + pb_ref[0]) + x1_ref[...] / d
        o_ref[...] = _linear_l2(x2, w2_ref[...], b2_ref[...], f_ref[0, 0])

    return pl.pallas_call(
        body,
        grid=(N // BM,),
        in_specs=[_PARTA, _PARTB, _ROW, _DEG,
                  _WFULL, _BIAS, _FLAG],
        out_specs=_ROW,
        out_shape=jax.ShapeDtypeStruct((N, D), jnp.float32),
    )(p2, p2, x1, deg, W2, b2, flag)


def kernel(h, edge_index, Norm, W0, b0, W1, b1, W2, b2):
    row1d = edge_index[0].astype(jnp.int32)
    col2d = edge_index[1].astype(jnp.int32).reshape(NW, CPW, G)
    flag = jnp.asarray(Norm, jnp.int32).reshape(1, 1)
    b0r = b0.reshape(1, D)
    b1r = b1.reshape(1, D)
    b2r = b2.reshape(1, D)

    degp = _sc_hist(col2d)
    deg = degp[0, :, :1] + degp[1, :, :1] + 1.0
    y0 = _dense1(h, deg)
    p1 = _sc_prop(row1d, col2d, y0)
    x1, y1 = _dense2(p1, h, deg)
    p2 = _sc_prop(row1d, col2d, y1)
    out01 = _final01(h, x1, W0, b0r, W1, b1r, flag)
    out2 = _final2(p2, x1, deg, W2, b2r, flag)
    return jnp.concatenate([out01, out2], axis=1)


# speedup vs baseline: 1.0112x; 1.0112x over previous
import functools

import jax
import jax.numpy as jnp
from jax import lax
from jax.experimental import pallas as pl
from jax.experimental.pallas import tpu as pltpu
from jax.experimental.pallas import tpu_sc as plsc

N = 10000
NP = 10240
E = 320000
D = 128
G = 80
NC = 2
NS = 16
NW = NC * NS
CPW = E // (G * NW)
RPS = NP // NS

BM = 1000


def _mesh():
    return plsc.VectorSubcoreMesh(core_axis_name="c", subcore_axis_name="s")


def _fill_rows(ref, val):
    v = jnp.full((16,), val, jnp.float32)
    rows = ref.shape[0]

    def body(i, c):
        for j in range(D // 16):
            ref[i, pl.ds(j * 16, 16)] = v
        return c

    lax.fori_loop(0, rows, body, 0)


def _zero_acc_async(zbuf, acc, sid, sem):
    for k2 in range(RPS // G):
        pltpu.async_copy(zbuf, acc.at[pl.ds(sid * RPS + k2 * G, G)], sem)


def _zero_acc_wait(zbuf, acc, sid, sem):
    for k2 in range(RPS // G):
        pltpu.make_async_copy(
            zbuf, acc.at[pl.ds(sid * RPS + k2 * G, G)], sem).wait()


def _sc_hist(col2d):
    @functools.partial(
        pl.kernel,
        out_type=jax.ShapeDtypeStruct((NC, NP, D), jnp.float32),
        mesh=_mesh(),
        scratch_types=[
            pltpu.VMEM((CPW, G), jnp.int32),
            pltpu.VMEM((G, D), jnp.float32),
            pltpu.VMEM((G, D), jnp.float32),
            pltpu.VMEM_SHARED((NP, D), jnp.float32),
            pltpu.SemaphoreType.DMA,
            pltpu.SemaphoreType.DMA,
        ],
    )
    def k(col_hbm, out_hbm, cidx_v, ones_v, zbuf, acc, semi, semz):
        cid = lax.axis_index("c")
        sid = lax.axis_index("s")
        wid = sid * NC + cid
        pltpu.async_copy(col_hbm.at[wid], cidx_v, semi)
        _fill_rows(zbuf, 0.0)
        _zero_acc_async(zbuf, acc, sid, semz)
        _fill_rows(ones_v, 1.0)
        pltpu.make_async_copy(col_hbm.at[wid], cidx_v, semi).wait()
        _zero_acc_wait(zbuf, acc, sid, semz)
        plsc.subcore_barrier()

        def body(g, c):
            pltpu.sync_copy(ones_v, acc.at[cidx_v.at[g]], add=True)
            return c

        lax.fori_loop(0, CPW, body, 0)
        plsc.subcore_barrier()
        pltpu.sync_copy(acc.at[pl.ds(sid * RPS, RPS)],
                        out_hbm.at[cid, pl.ds(sid * RPS, RPS)])

    return k(col2d)


def _sc_prop(row1d, col2d, y):
    @functools.partial(
        pl.kernel,
        out_type=jax.ShapeDtypeStruct((NC, NP, D), jnp.float32),
        mesh=_mesh(),
        scratch_types=[
            pltpu.VMEM((CPW * G,), jnp.int32),
            pltpu.VMEM((CPW, G), jnp.int32),
            pltpu.VMEM((G, D), jnp.float32),
            pltpu.VMEM((G, D), jnp.float32),
            pltpu.VMEM_SHARED((NP, D), jnp.float32),
            pltpu.SemaphoreType.DMA,
            pltpu.SemaphoreType.DMA,
            pltpu.SemaphoreType.DMA,
        ],
    )
    def k(row_hbm, col_hbm, y_hbm, out_hbm,
          ridx_v, cidx_v, buf0, buf1, acc, sem0, sem1, semz):
        cid = lax.axis_index("c")
        sid = lax.axis_index("s")
        wid = sid * NC + cid
        pltpu.async_copy(row_hbm.at[pl.ds(wid * (CPW * G), CPW * G)],
                         ridx_v, sem0)
        pltpu.async_copy(col_hbm.at[wid], cidx_v, sem1)
        _fill_rows(buf0, 0.0)
        _zero_acc_async(buf0, acc, sid, semz)
        pltpu.make_async_copy(row_hbm.at[pl.ds(wid * (CPW * G), CPW * G)],
                              ridx_v, sem0).wait()
        pltpu.make_async_copy(col_hbm.at[wid], cidx_v, sem1).wait()
        _zero_acc_wait(buf0, acc, sid, semz)
        plsc.subcore_barrier()

        H = G // 2

        def gfire(g, buf, sem):
            pltpu.async_copy(y_hbm.at[ridx_v.at[pl.ds(g * G, H)]],
                             buf.at[pl.ds(0, H)], sem)
            pltpu.async_copy(y_hbm.at[ridx_v.at[pl.ds(g * G + H, H)]],
                             buf.at[pl.ds(H, H)], sem)

        def gwait(g, buf, sem):
            pltpu.make_async_copy(y_hbm.at[ridx_v.at[pl.ds(g * G, H)]],
                                  buf.at[pl.ds(0, H)], sem).wait()
            pltpu.make_async_copy(y_hbm.at[ridx_v.at[pl.ds(g * G + H, H)]],
                                  buf.at[pl.ds(H, H)], sem).wait()

        gfire(0, buf0, sem0)

        def body(gg, c):
            g = 2 * gg
            gwait(g, buf0, sem0)
            gfire(g + 1, buf1, sem1)
            pltpu.sync_copy(buf0, acc.at[cidx_v.at[g]], add=True)
            gwait(g + 1, buf1, sem1)
            gfire(g + 2, buf0, sem0)
            pltpu.sync_copy(buf1, acc.at[cidx_v.at[g + 1]], add=True)
            return c

        lax.fori_loop(0, CPW // 2, body, 0)
        gwait(CPW - 1, buf0, sem0)
        pltpu.sync_copy(buf0, acc.at[cidx_v.at[CPW - 1]], add=True)
        plsc.subcore_barrier()
        pltpu.sync_copy(acc.at[pl.ds(sid * RPS, RPS)],
                        out_hbm.at[cid, pl.ds(sid * RPS, RPS)])

    return k(row1d, col2d, y)


_ROW = pl.BlockSpec((BM, D), lambda i: (i, 0))
_PARTA = pl.BlockSpec((1, BM, D), lambda i: (0, i, 0))
_PARTB = pl.BlockSpec((1, BM, D), lambda i: (1, i, 0))
_DEGA = pl.BlockSpec((1, BM, D), lambda i: (0, i, 0))
_DEGB = pl.BlockSpec((1, BM, D), lambda i: (1, i, 0))
_WFULL = pl.BlockSpec((D, D), lambda i: (0, 0))
_BIAS = pl.BlockSpec((1, D), lambda i: (0, 0))
_FLAG = pl.BlockSpec((1, 1), lambda i: (0, 0), memory_space=pltpu.SMEM)


def _deg_of(da_ref, db_ref):
    return da_ref[0][:, :1] + db_ref[0][:, :1] + 1.0


def _linear_l2(m, w, b, f):
    t = lax.dot_general(m, w, (((1,), (1,)), ((), ())),
                        preferred_element_type=jnp.float32) + b
    nrm = jnp.sqrt(jnp.sum(t * t, axis=1, keepdims=True))
    tn = t / jnp.maximum(nrm, 1e-12)
    return jnp.where(f == 1, tn, t)


def _dense1(h, degp):
    def body(h_ref, da_ref, db_ref, o_ref):
        o_ref[...] = h_ref[...] * lax.rsqrt(_deg_of(da_ref, db_ref))

    return pl.pallas_call(
        body,
        grid=(N // BM,),
        in_specs=[_ROW, _DEGA, _DEGB],
        out_specs=_ROW,
        out_shape=jax.ShapeDtypeStruct((N, D), jnp.float32),
    )(h, degp, degp)


def _dense2(p1, h, degp):
    def body(pa_ref, pb_ref, h_ref, da_ref, db_ref, xo_ref, yo_ref):
        d = _deg_of(da_ref, db_ref)
        s = lax.rsqrt(d)
        x = s * (pa_ref[0] + pb_ref[0]) + h_ref[...] / d
        xo_ref[...] = x
        yo_ref[...] = s * x

    return pl.pallas_call(
        body,
        grid=(N // BM,),
        in_specs=[_PARTA, _PARTB, _ROW, _DEGA, _DEGB],
        out_specs=[_ROW, _ROW],
        out_shape=[jax.ShapeDtypeStruct((N, D), jnp.float32),
                   jax.ShapeDtypeStruct((N, D), jnp.float32)],
    )(p1, p1, h, degp, degp)


def _final01(h, x1, W0, b0, W1, b1, flag):
    def body(h_ref, x1_ref, w0_ref, b0_ref, w1_ref, b1_ref, f_ref, o_ref):
        f = f_ref[0, 0]
        o_ref[:, 0:D] = _linear_l2(h_ref[...], w0_ref[...], b0_ref[...], f)
        o_ref[:, D:2 * D] = _linear_l2(x1_ref[...], w1_ref[...],
                                       b1_ref[...], f)

    return pl.pallas_call(
        body,
        grid=(N // BM,),
        in_specs=[_ROW, _ROW, _WFULL, _BIAS, _WFULL, _BIAS, _FLAG],
        out_specs=pl.BlockSpec((BM, 2 * D), lambda i: (i, 0)),
        out_shape=jax.ShapeDtypeStruct((N, 2 * D), jnp.float32),
    )(h, x1, W0, b0, W1, b1, flag)


def _final2(p2, x1, degp, W2, b2, flag):
    def body(pa_ref, pb_ref, x1_ref, da_ref, db_ref,
             w2_ref, b2_ref, f_ref, o_ref):
        d = _deg_of(da_ref, db_ref)
        s = lax.rsqrt(d)
        x2 = s * (pa_ref[0] + pb_ref[0]) + x1_ref[...] / d
        o_ref[...] = _linear_l2(x2, w2_ref[...], b2_ref[...], f_ref[0, 0])

    return pl.pallas_call(
        body,
        grid=(N // BM,),
        in_specs=[_PARTA, _PARTB, _ROW, _DEGA, _DEGB,
                  _WFULL, _BIAS, _FLAG],
        out_specs=_ROW,
        out_shape=jax.ShapeDtypeStruct((N, D), jnp.float32),
    )(p2, p2, x1, degp, degp, W2, b2, flag)


def kernel(h, edge_index, Norm, W0, b0, W1, b1, W2, b2):
    row1d = edge_index[0].astype(jnp.int32)
    col2d = edge_index[1].astype(jnp.int32).reshape(NW, CPW, G)
    flag = jnp.asarray(Norm, jnp.int32).reshape(1, 1)
    b0r = b0.reshape(1, D)
    b1r = b1.reshape(1, D)
    b2r = b2.reshape(1, D)

    degp = _sc_hist(col2d)
    y0 = _dense1(h, degp)
    p1 = _sc_prop(row1d, col2d, y0)
    x1, y1 = _dense2(p1, h, degp)
    p2 = _sc_prop(row1d, col2d, y1)
    out01 = _final01(h, x1, W0, b0r, W1, b1r, flag)
    out2 = _final2(p2, x1, degp, W2, b2r, flag)
    return jnp.concatenate([out01, out2], axis=1)


# speedup vs baseline: 1.0126x; 1.0014x over previous
import functools

import jax
import jax.numpy as jnp
from jax import lax
from jax.experimental import pallas as pl
from jax.experimental.pallas import tpu as pltpu
from jax.experimental.pallas import tpu_sc as plsc

N = 10000
NP = 10240
E = 320000
D = 128
G = 80
NC = 2
NS = 16
NW = NC * NS
CPW = E // (G * NW)
RPS = NP // NS

BM = 1000


def _mesh():
    return plsc.VectorSubcoreMesh(core_axis_name="c", subcore_axis_name="s")


def _fill_rows(ref, val):
    v = jnp.full((16,), val, jnp.float32)
    rows = ref.shape[0]

    def body(i, c):
        for j in range(D // 16):
            ref[i, pl.ds(j * 16, 16)] = v
        return c

    lax.fori_loop(0, rows, body, 0)


def _zero_acc_async(zbuf, acc, sid, sem):
    for k2 in range(RPS // G):
        pltpu.async_copy(zbuf, acc.at[pl.ds(sid * RPS + k2 * G, G)], sem)


def _zero_acc_wait(zbuf, acc, sid, sem):
    for k2 in range(RPS // G):
        pltpu.make_async_copy(
            zbuf, acc.at[pl.ds(sid * RPS + k2 * G, G)], sem).wait()


def _sc_hist(col2d):
    @functools.partial(
        pl.kernel,
        out_type=jax.ShapeDtypeStruct((NC, NP, D), jnp.float32),
        mesh=_mesh(),
        scratch_types=[
            pltpu.VMEM((CPW, G), jnp.int32),
            pltpu.VMEM((G, D), jnp.float32),
            pltpu.VMEM((G, D), jnp.float32),
            pltpu.VMEM_SHARED((NP, D), jnp.float32),
            pltpu.SemaphoreType.DMA,
            pltpu.SemaphoreType.DMA,
        ],
    )
    def k(col_hbm, out_hbm, cidx_v, ones_v, zbuf, acc, semi, semz):
        cid = lax.axis_index("c")
        sid = lax.axis_index("s")
        wid = sid * NC + cid
        pltpu.async_copy(col_hbm.at[wid], cidx_v, semi)
        _fill_rows(zbuf, 0.0)
        _zero_acc_async(zbuf, acc, sid, semz)
        _fill_rows(ones_v, 1.0)
        pltpu.make_async_copy(col_hbm.at[wid], cidx_v, semi).wait()
        _zero_acc_wait(zbuf, acc, sid, semz)
        plsc.subcore_barrier()

        pltpu.async_copy(ones_v, acc.at[cidx_v.at[0]], semz, add=True)

        def body(g, c):
            pltpu.async_copy(ones_v, acc.at[cidx_v.at[g + 1]], semz,
                             add=True)
            pltpu.make_async_copy(ones_v, acc.at[cidx_v.at[g]], semz).wait()
            return c

        lax.fori_loop(0, CPW - 1, body, 0)
        pltpu.make_async_copy(ones_v, acc.at[cidx_v.at[CPW - 1]],
                              semz).wait()
        plsc.subcore_barrier()
        pltpu.sync_copy(acc.at[pl.ds(sid * RPS, RPS)],
                        out_hbm.at[cid, pl.ds(sid * RPS, RPS)])

    return k(col2d)


def _sc_prop(row1d, col2d, y):
    @functools.partial(
        pl.kernel,
        out_type=jax.ShapeDtypeStruct((NC, NP, D), jnp.float32),
        mesh=_mesh(),
        scratch_types=[
            pltpu.VMEM((CPW * G,), jnp.int32),
            pltpu.VMEM((CPW, G), jnp.int32),
            pltpu.VMEM((G, D), jnp.float32),
            pltpu.VMEM((G, D), jnp.float32),
            pltpu.VMEM_SHARED((NP, D), jnp.float32),
            pltpu.SemaphoreType.DMA,
            pltpu.SemaphoreType.DMA,
            pltpu.SemaphoreType.DMA,
        ],
    )
    def k(row_hbm, col_hbm, y_hbm, out_hbm,
          ridx_v, cidx_v, buf0, buf1, acc, sem0, sem1, semz):
        cid = lax.axis_index("c")
        sid = lax.axis_index("s")
        wid = sid * NC + cid
        pltpu.async_copy(row_hbm.at[pl.ds(wid * (CPW * G), CPW * G)],
                         ridx_v, sem0)
        pltpu.async_copy(col_hbm.at[wid], cidx_v, sem1)
        _fill_rows(buf0, 0.0)
        _zero_acc_async(buf0, acc, sid, semz)
        pltpu.make_async_copy(row_hbm.at[pl.ds(wid * (CPW * G), CPW * G)],
                              ridx_v, sem0).wait()
        pltpu.make_async_copy(col_hbm.at[wid], cidx_v, sem1).wait()
        _zero_acc_wait(buf0, acc, sid, semz)
        plsc.subcore_barrier()

        H = G // 2

        def gfire(g, buf, sem):
            pltpu.async_copy(y_hbm.at[ridx_v.at[pl.ds(g * G, H)]],
                             buf.at[pl.ds(0, H)], sem)
            pltpu.async_copy(y_hbm.at[ridx_v.at[pl.ds(g * G + H, H)]],
                             buf.at[pl.ds(H, H)], sem)

        def gwait(g, buf, sem):
            pltpu.make_async_copy(y_hbm.at[ridx_v.at[pl.ds(g * G, H)]],
                                  buf.at[pl.ds(0, H)], sem).wait()
            pltpu.make_async_copy(y_hbm.at[ridx_v.at[pl.ds(g * G + H, H)]],
                                  buf.at[pl.ds(H, H)], sem).wait()

        gfire(0, buf0, sem0)

        def body(gg, c):
            g = 2 * gg
            gwait(g, buf0, sem0)
            gfire(g + 1, buf1, sem1)
            pltpu.sync_copy(buf0, acc.at[cidx_v.at[g]], add=True)
            gwait(g + 1, buf1, sem1)
            gfire(g + 2, buf0, sem0)
            pltpu.sync_copy(buf1, acc.at[cidx_v.at[g + 1]], add=True)
            return c

        lax.fori_loop(0, CPW // 2, body, 0)
        gwait(CPW - 1, buf0, sem0)
        pltpu.sync_copy(buf0, acc.at[cidx_v.at[CPW - 1]], add=True)
        plsc.subcore_barrier()
        pltpu.sync_copy(acc.at[pl.ds(sid * RPS, RPS)],
                        out_hbm.at[cid, pl.ds(sid * RPS, RPS)])

    return k(row1d, col2d, y)


_ROW = pl.BlockSpec((BM, D), lambda i: (i, 0))
_PARTA = pl.BlockSpec((1, BM, D), lambda i: (0, i, 0))
_PARTB = pl.BlockSpec((1, BM, D), lambda i: (1, i, 0))
_DEGA = pl.BlockSpec((1, BM, D), lambda i: (0, i, 0))
_DEGB = pl.BlockSpec((1, BM, D), lambda i: (1, i, 0))
_WFULL = pl.BlockSpec((D, D), lambda i: (0, 0))
_BIAS = pl.BlockSpec((1, D), lambda i: (0, 0))
_FLAG = pl.BlockSpec((1, 1), lambda i: (0, 0), memory_space=pltpu.SMEM)


def _deg_of(da_ref, db_ref):
    return da_ref[0][:, :1] + db_ref[0][:, :1] + 1.0


def _linear_l2(m, w, b, f):
    t = lax.dot_general(m, w, (((1,), (1,)), ((), ())),
                        preferred_element_type=jnp.float32) + b
    nrm = jnp.sqrt(jnp.sum(t * t, axis=1, keepdims=True))
    tn = t / jnp.maximum(nrm, 1e-12)
    return jnp.where(f == 1, tn, t)


def _dense1(h, degp):
    def body(h_ref, da_ref, db_ref, o_ref):
        o_ref[...] = h_ref[...] * lax.rsqrt(_deg_of(da_ref, db_ref))

    return pl.pallas_call(
        body,
        grid=(N // BM,),
        in_specs=[_ROW, _DEGA, _DEGB],
        out_specs=_ROW,
        out_shape=jax.ShapeDtypeStruct((N, D), jnp.float32),
    )(h, degp, degp)


def _dense2(p1, h, degp):
    def body(pa_ref, pb_ref, h_ref, da_ref, db_ref, xo_ref, yo_ref):
        d = _deg_of(da_ref, db_ref)
        s = lax.rsqrt(d)
        x = s * (pa_ref[0] + pb_ref[0]) + h_ref[...] / d
        xo_ref[...] = x
        yo_ref[...] = s * x

    return pl.pallas_call(
        body,
        grid=(N // BM,),
        in_specs=[_PARTA, _PARTB, _ROW, _DEGA, _DEGB],
        out_specs=[_ROW, _ROW],
        out_shape=[jax.ShapeDtypeStruct((N, D), jnp.float32),
                   jax.ShapeDtypeStruct((N, D), jnp.float32)],
    )(p1, p1, h, degp, degp)


def _final01(h, x1, W0, b0, W1, b1, flag):
    def body(h_ref, x1_ref, w0_ref, b0_ref, w1_ref, b1_ref, f_ref, o_ref):
        f = f_ref[0, 0]
        o_ref[:, 0:D] = _linear_l2(h_ref[...], w0_ref[...], b0_ref[...], f)
        o_ref[:, D:2 * D] = _linear_l2(x1_ref[...], w1_ref[...],
                                       b1_ref[...], f)

    return pl.pallas_call(
        body,
        grid=(N // BM,),
        in_specs=[_ROW, _ROW, _WFULL, _BIAS, _WFULL, _BIAS, _FLAG],
        out_specs=pl.BlockSpec((BM, 2 * D), lambda i: (i, 0)),
        out_shape=jax.ShapeDtypeStruct((N, 2 * D), jnp.float32),
    )(h, x1, W0, b0, W1, b1, flag)


def _final2(p2, x1, degp, W2, b2, flag):
    def body(pa_ref, pb_ref, x1_ref, da_ref, db_ref,
             w2_ref, b2_ref, f_ref, o_ref):
        d = _deg_of(da_ref, db_ref)
        s = lax.rsqrt(d)
        x2 = s * (pa_ref[0] + pb_ref[0]) + x1_ref[...] / d
        o_ref[...] = _linear_l2(x2, w2_ref[...], b2_ref[...], f_ref[0, 0])

    return pl.pallas_call(
        body,
        grid=(N // BM,),
        in_specs=[_PARTA, _PARTB, _ROW, _DEGA, _DEGB,
                  _WFULL, _BIAS, _FLAG],
        out_specs=_ROW,
        out_shape=jax.ShapeDtypeStruct((N, D), jnp.float32),
    )(p2, p2, x1, degp, degp, W2, b2, flag)


def kernel(h, edge_index, Norm, W0, b0, W1, b1, W2, b2):
    row1d = edge_index[0].astype(jnp.int32)
    col2d = edge_index[1].astype(jnp.int32).reshape(NW, CPW, G)
    flag = jnp.asarray(Norm, jnp.int32).reshape(1, 1)
    b0r = b0.reshape(1, D)
    b1r = b1.reshape(1, D)
    b2r = b2.reshape(1, D)

    degp = _sc_hist(col2d)
    y0 = _dense1(h, degp)
    p1 = _sc_prop(row1d, col2d, y0)
    x1, y1 = _dense2(p1, h, degp)
    p2 = _sc_prop(row1d, col2d, y1)
    out01 = _final01(h, x1, W0, b0r, W1, b1r, flag)
    out2 = _final2(p2, x1, degp, W2, b2r, flag)
    return jnp.concatenate([out01, out2], axis=1)
